# Initial kernel scaffold; baseline (speedup 1.0000x reference)
import jax
import jax.numpy as jnp
from jax.experimental import pallas as pl


def kernel(x_author, x_paper, params, ei_writes, ei_rev, ei_cites):
    raise NotImplementedError("write your pallas kernel here")


# speedup vs baseline: 2.3569x; 2.3569x over previous
import functools
import jax
import jax.numpy as jnp
from jax import lax
from jax.experimental import pallas as pl
from jax.experimental.pallas import tpu as pltpu
from jax.experimental.pallas import tpu_sc as plsc

N = 10000
D = 128
H = 128
L_OUT = 8
EPS = 1e-05
E = 160000

NC = 2
NS = 16
NW = NC * NS

C = 128
EP = 5120
NCH = EP // C
NACC = 10240
RPT = NACC // NS
ZR = 32
GB = 1000
G = N // GB


def _seg_body(with_counts, xa_h, xp_h, src_h, dst_h, *rest):
  if with_counts:
    (sums_o, cnt_o, accum, sidx, didx, rows, zrow, gsem) = rest
  else:
    (sums_o, accum, sidx, didx, rows, zrow, gsem) = rest
    cnt_o = None

  cid = lax.axis_index("c")
  sid = lax.axis_index("s")
  wid = cid * NS + sid

  zero16 = jnp.zeros((16,), jnp.float32)

  def zfill(i, _):
    zrow[i // 8, pl.ds((i % 8) * 16, 16)] = zero16
    return 0
  lax.fori_loop(0, ZR * (D // 16), zfill, 0)

  tables = (xa_h, xp_h, xp_h)
  for t in range(3):
    table = tables[t]
    for k in range(RPT // ZR):
      pltpu.sync_copy(zrow, accum.at[pl.ds(sid * RPT + k * ZR, ZR)])
    plsc.subcore_barrier()

    pltpu.sync_copy(src_h.at[t, wid], sidx)
    pltpu.sync_copy(dst_h.at[t, wid], didx)

    def chunk(j, _):
      pltpu.async_copy(table.at[sidx.at[j]], rows, gsem).wait()
      pltpu.sync_copy(rows, accum.at[didx.at[j]], add=True)
      return 0
    lax.fori_loop(0, NCH, chunk, 0)
    plsc.subcore_barrier()

    pltpu.sync_copy(accum.at[pl.ds(sid * RPT, RPT)],
                    sums_o.at[t, cid, pl.ds(sid * RPT, RPT)])

  if with_counts:
    one16 = jnp.ones((16,), jnp.float32)
    def ofill(i, _):
      rows[i // 8, pl.ds((i % 8) * 16, 16)] = one16
      return 0
    lax.fori_loop(0, C * (D // 16), ofill, 0)

    for t in range(3):
      for k in range(RPT // ZR):
        pltpu.sync_copy(zrow, accum.at[pl.ds(sid * RPT + k * ZR, ZR)])
      plsc.subcore_barrier()

      pltpu.sync_copy(dst_h.at[t, wid], didx)

      def cchunk(j, _):
        pltpu.sync_copy(rows, accum.at[didx.at[j]], add=True)
        return 0
      lax.fori_loop(0, NCH, cchunk, 0)
      plsc.subcore_barrier()

      pltpu.sync_copy(accum.at[pl.ds(sid * RPT, RPT)],
                      cnt_o.at[t, cid, pl.ds(sid * RPT, RPT)])


def _make_seg_call(with_counts):
  out_type = [jax.ShapeDtypeStruct((3, NC, NACC, D), jnp.float32)]
  if with_counts:
    out_type.append(jax.ShapeDtypeStruct((3, NC, NACC, D), jnp.float32))
  return pl.kernel(
      functools.partial(_seg_body, with_counts),
      out_type=tuple(out_type),
      mesh=plsc.VectorSubcoreMesh(core_axis_name="c", subcore_axis_name="s",
                                  num_cores=NC, num_subcores=NS),
      scratch_types=(
          pltpu.VMEM_SHARED((NACC, D), jnp.float32),
          pltpu.VMEM((NCH, C), jnp.int32),
          pltpu.VMEM((NCH, C), jnp.int32),
          pltpu.VMEM((C, D), jnp.float32),
          pltpu.VMEM((ZR, D), jnp.float32),
          pltpu.SemaphoreType.DMA,
      ),
      name="seg_sum_counts" if with_counts else "seg_sum",
  )


def _sage_block(xd, aggr, wd, bd, ws, bs, wu, bu):
  dst_msg = jnp.dot(xd, wd, preferred_element_type=jnp.float32) + bd
  src_msg = jnp.dot(aggr, ws, preferred_element_type=jnp.float32) + bs
  return (jnp.dot(dst_msg, wu[:H], preferred_element_type=jnp.float32)
          + jnp.dot(src_msg, wu[H:], preferred_element_type=jnp.float32) + bu)


def _dense_a_kernel(xa, xp, sums, cnt, wd, bd, ws, bs, wu, bu,
                    hap, hpp, stats):
  xa_b = xa[...]
  xp_b = xp[...]
  m = []
  for t in range(3):
    s_t = sums[t, 0] + sums[t, 1]
    c_t = cnt[t, 0, :, 0:1] + cnt[t, 1, :, 0:1]
    aggr = s_t / jnp.clip(c_t, 1.0, None)
    xd = xa_b if t == 1 else xp_b
    m.append(_sage_block(xd, aggr, wd[t], bd[t], ws[t], bs[t], wu[t], bu[t]))
  ha = m[1]
  hp = 0.5 * (m[0] + m[2])
  hap[...] = ha
  hpp[...] = hp
  stats[0, 0, 0] = jnp.sum(ha, axis=0)
  stats[0, 0, 1] = jnp.sum(ha * ha, axis=0)
  stats[0, 1, 0] = jnp.sum(hp, axis=0)
  stats[0, 1, 1] = jnp.sum(hp * hp, axis=0)


def _bn_act(x, st, gamma, beta):
  mu = jnp.sum(st[:, 0], axis=0) / N
  var = jnp.sum(st[:, 1], axis=0) / N - mu * mu
  xn = (x - mu) / jnp.sqrt(var + EPS) * gamma + beta
  return jnp.where(xn >= 0, xn, 0.01 * xn)


def _dense_b_kernel(hap, hpp, stats, ga, ba, gp, bp, ha_o, hp_o):
  st = stats[...]
  ha_o[...] = _bn_act(hap[...], st[:, 0], ga[0], ba[0])
  hp_o[...] = _bn_act(hpp[...], st[:, 1], gp[0], bp[0])


def _dense_b2_kernel(hap, hpp, stats, ga, ba, gp, bp, wpa, bpa, wpp, bpp,
                     out_o):
  st = stats[...]
  ha = _bn_act(hap[...], st[:, 0], ga[0], ba[0])
  hp = _bn_act(hpp[...], st[:, 1], gp[0], bp[0])
  out_o[0] = jnp.dot(ha, wpa[...], preferred_element_type=jnp.float32) + bpa[0]
  out_o[1] = jnp.dot(hp, wpp[...], preferred_element_type=jnp.float32) + bpp[0]


def _row_spec(shape):
  nd = len(shape)
  blk = tuple(shape[:-2]) + (GB, shape[-1])
  def im(i):
    return tuple([0] * (nd - 2) + [i, 0])
  return pl.BlockSpec(blk, im)


def _full_spec(shape):
  nd = len(shape)
  return pl.BlockSpec(tuple(shape), lambda i, nd=nd: (0,) * nd)


def _dense_a(xa, xp, sums, cnt, wd, bd, ws, bs, wu, bu):
  in_arrs = (xa, xp, sums, cnt, wd, bd, ws, bs, wu, bu)
  in_specs = [
      _row_spec(xa.shape), _row_spec(xp.shape),
      _row_spec(sums.shape), _row_spec(cnt.shape),
      _full_spec(wd.shape), _full_spec(bd.shape),
      _full_spec(ws.shape), _full_spec(bs.shape),
      _full_spec(wu.shape), _full_spec(bu.shape),
  ]
  out_shape = (
      jax.ShapeDtypeStruct((N, H), jnp.float32),
      jax.ShapeDtypeStruct((N, H), jnp.float32),
      jax.ShapeDtypeStruct((G, 2, 2, H), jnp.float32),
  )
  out_specs = (
      _row_spec((N, H)), _row_spec((N, H)),
      pl.BlockSpec((1, 2, 2, H), lambda i: (i, 0, 0, 0)),
  )
  return pl.pallas_call(
      _dense_a_kernel, grid=(G,), in_specs=in_specs,
      out_specs=out_specs, out_shape=out_shape)(*in_arrs)


def _dense_b(hap, hpp, stats, ga, ba, gp, bp):
  in_specs = [
      _row_spec((N, H)), _row_spec((N, H)), _full_spec(stats.shape),
      _full_spec(ga.shape), _full_spec(ba.shape),
      _full_spec(gp.shape), _full_spec(bp.shape),
  ]
  out_shape = (
      jax.ShapeDtypeStruct((N, H), jnp.float32),
      jax.ShapeDtypeStruct((N, H), jnp.float32),
  )
  out_specs = (_row_spec((N, H)), _row_spec((N, H)))
  return pl.pallas_call(
      _dense_b_kernel, grid=(G,), in_specs=in_specs,
      out_specs=out_specs, out_shape=out_shape)(hap, hpp, stats, ga, ba, gp, bp)


def _dense_b2(hap, hpp, stats, ga, ba, gp, bp, wpa, bpa, wpp, bpp):
  in_specs = [
      _row_spec((N, H)), _row_spec((N, H)), _full_spec(stats.shape),
      _full_spec(ga.shape), _full_spec(ba.shape),
      _full_spec(gp.shape), _full_spec(bp.shape),
      _full_spec(wpa.shape), _full_spec(bpa.shape),
      _full_spec(wpp.shape), _full_spec(bpp.shape),
  ]
  out_shape = jax.ShapeDtypeStruct((2, N, L_OUT), jnp.float32)
  out_specs = pl.BlockSpec((2, GB, L_OUT), lambda i: (0, i, 0))
  return pl.pallas_call(
      _dense_b2_kernel, grid=(G,), in_specs=in_specs,
      out_specs=out_specs, out_shape=out_shape)(
          hap, hpp, stats, ga, ba, gp, bp, wpa, bpa, wpp, bpp)


def _prep_edges(ei):
  pad = NW * EP - E
  s = jnp.concatenate([ei[0], jnp.zeros((pad,), jnp.int32)])
  d = jnp.concatenate([ei[1], jnp.full((pad,), N, jnp.int32)])
  return s.reshape(NW, NCH, C), d.reshape(NW, NCH, C)


def _stack_sage(p_layer):
  wd = jnp.stack([p_layer[k]["dst"]["W"] for k in ("writes", "rev", "cites")])
  bd = jnp.stack([p_layer[k]["dst"]["b"][None] for k in ("writes", "rev", "cites")])
  ws = jnp.stack([p_layer[k]["src"]["W"] for k in ("writes", "rev", "cites")])
  bs = jnp.stack([p_layer[k]["src"]["b"][None] for k in ("writes", "rev", "cites")])
  wu = jnp.stack([p_layer[k]["upd"]["W"] for k in ("writes", "rev", "cites")])
  bu = jnp.stack([p_layer[k]["upd"]["b"][None] for k in ("writes", "rev", "cites")])
  return wd, bd, ws, bs, wu, bu


def kernel(x_author, x_paper, params, ei_writes, ei_rev, ei_cites):
  sw, dw = _prep_edges(ei_writes)
  sr, dr = _prep_edges(ei_rev)
  sc, dc = _prep_edges(ei_cites)
  src = jnp.stack([sw, sr, sc])
  dst = jnp.stack([dw, dr, dc])

  sums1, cnt = _make_seg_call(True)(x_author, x_paper, src, dst)

  w1 = _stack_sage(params["l1"])
  bn1 = params["bn1"]
  hap1, hpp1, st1 = _dense_a(x_author, x_paper, sums1, cnt, *w1)
  ha1, hp1 = _dense_b(hap1, hpp1, st1,
                      bn1["author"]["gamma"][None], bn1["author"]["beta"][None],
                      bn1["paper"]["gamma"][None], bn1["paper"]["beta"][None])

  (sums2,) = _make_seg_call(False)(ha1, hp1, src, dst)

  w2 = _stack_sage(params["l2"])
  bn2 = params["bn2"]
  hap2, hpp2, st2 = _dense_a(ha1, hp1, sums2, cnt, *w2)
  out = _dense_b2(hap2, hpp2, st2,
                  bn2["author"]["gamma"][None], bn2["author"]["beta"][None],
                  bn2["paper"]["gamma"][None], bn2["paper"]["beta"][None],
                  params["post"]["author"]["W"], params["post"]["author"]["b"][None],
                  params["post"]["paper"]["W"], params["post"]["paper"]["b"][None])
  return out


# speedup vs baseline: 2.3599x; 1.0013x over previous
import functools
import jax
import jax.numpy as jnp
from jax import lax
from jax.experimental import pallas as pl
from jax.experimental.pallas import tpu as pltpu
from jax.experimental.pallas import tpu_sc as plsc

N = 10000
D = 128
H = 128
L_OUT = 8
EPS = 1e-05
E = 160000

NC = 2
NS = 16
NW = NC * NS

C = 64
EP = 5120
NCH = EP // C
NPAIR = NCH // 2
NACC = 10240
RPT = NACC // NS
ZR = 32
GB = 1000
G = N // GB


def _seg_body(with_counts, xa_h, xp_h, src_h, dst_h, *rest):
  if with_counts:
    (sums_o, cnt_o, accum, sidx, didx, rows, zrow, g0, g1) = rest
  else:
    (sums_o, accum, sidx, didx, rows, zrow, g0, g1) = rest
    cnt_o = None

  cid = lax.axis_index("c")
  sid = lax.axis_index("s")
  wid = cid * NS + sid

  zero16 = jnp.zeros((16,), jnp.float32)

  def zfill(i, _):
    zrow[i // 8, pl.ds((i % 8) * 16, 16)] = zero16
    return 0
  lax.fori_loop(0, ZR * (D // 16), zfill, 0)

  tables = (xa_h, xp_h, xp_h)
  for t in range(3):
    table = tables[t]
    for k in range(RPT // ZR):
      pltpu.sync_copy(zrow, accum.at[pl.ds(sid * RPT + k * ZR, ZR)])
    plsc.subcore_barrier()

    pltpu.sync_copy(src_h.at[t, wid], sidx)
    pltpu.sync_copy(dst_h.at[t, wid], didx)

    pltpu.async_copy(table.at[sidx.at[0]], rows.at[0], g0)

    def chunk_pair(p, _):
      j0 = 2 * p
      pltpu.make_async_copy(table.at[sidx.at[j0]], rows.at[0], g0).wait()
      pltpu.async_copy(table.at[sidx.at[j0 + 1]], rows.at[1], g1)
      pltpu.sync_copy(rows.at[0], accum.at[didx.at[j0]], add=True)
      pltpu.make_async_copy(table.at[sidx.at[j0 + 1]], rows.at[1], g1).wait()
      @pl.when(p + 1 < NPAIR)
      def _():
        pltpu.async_copy(table.at[sidx.at[j0 + 2]], rows.at[0], g0)
      pltpu.sync_copy(rows.at[1], accum.at[didx.at[j0 + 1]], add=True)
      return 0
    lax.fori_loop(0, NPAIR, chunk_pair, 0)
    plsc.subcore_barrier()

    pltpu.sync_copy(accum.at[pl.ds(sid * RPT, RPT)],
                    sums_o.at[t, cid, pl.ds(sid * RPT, RPT)])

  if with_counts:
    one16 = jnp.ones((16,), jnp.float32)
    def ofill(i, _):
      rows[0, i // 8, pl.ds((i % 8) * 16, 16)] = one16
      return 0
    lax.fori_loop(0, C * (D // 16), ofill, 0)

    for t in range(3):
      for k in range(RPT // ZR):
        pltpu.sync_copy(zrow, accum.at[pl.ds(sid * RPT + k * ZR, ZR)])
      plsc.subcore_barrier()

      pltpu.sync_copy(dst_h.at[t, wid], didx)

      K = 8
      def cbatch(b, _):
        for u in range(K):
          pltpu.async_copy(rows.at[0], accum.at[didx.at[b * K + u]], g0,
                           add=True)
        for u in range(K):
          pltpu.make_async_copy(rows.at[0], accum.at[didx.at[b * K + u]],
                                g0).wait()
        return 0
      lax.fori_loop(0, NCH // K, cbatch, 0)
      plsc.subcore_barrier()

      pltpu.sync_copy(accum.at[pl.ds(sid * RPT, RPT)],
                      cnt_o.at[t, cid, pl.ds(sid * RPT, RPT)])


def _make_seg_call(with_counts):
  out_type = [jax.ShapeDtypeStruct((3, NC, NACC, D), jnp.float32)]
  if with_counts:
    out_type.append(jax.ShapeDtypeStruct((3, NC, NACC, D), jnp.float32))
  return pl.kernel(
      functools.partial(_seg_body, with_counts),
      out_type=tuple(out_type),
      mesh=plsc.VectorSubcoreMesh(core_axis_name="c", subcore_axis_name="s",
                                  num_cores=NC, num_subcores=NS),
      scratch_types=(
          pltpu.VMEM_SHARED((NACC, D), jnp.float32),
          pltpu.VMEM((NCH, C), jnp.int32),
          pltpu.VMEM((NCH, C), jnp.int32),
          pltpu.VMEM((2, C, D), jnp.float32),
          pltpu.VMEM((ZR, D), jnp.float32),
          pltpu.SemaphoreType.DMA,
          pltpu.SemaphoreType.DMA,
      ),
      name="seg_sum_counts" if with_counts else "seg_sum",
  )


def _sage_block(xd, aggr, wd, bd, ws, bs, wu, bu):
  dst_msg = jnp.dot(xd, wd, preferred_element_type=jnp.float32) + bd
  src_msg = jnp.dot(aggr, ws, preferred_element_type=jnp.float32) + bs
  return (jnp.dot(dst_msg, wu[:H], preferred_element_type=jnp.float32)
          + jnp.dot(src_msg, wu[H:], preferred_element_type=jnp.float32) + bu)


def _dense_a_kernel(xa, xp, sums, cnt, wd, bd, ws, bs, wu, bu,
                    hap, hpp, stats):
  xa_b = xa[...]
  xp_b = xp[...]
  m = []
  for t in range(3):
    s_t = sums[t, 0] + sums[t, 1]
    c_t = cnt[t, 0, :, 0:1] + cnt[t, 1, :, 0:1]
    aggr = s_t / jnp.clip(c_t, 1.0, None)
    xd = xa_b if t == 1 else xp_b
    m.append(_sage_block(xd, aggr, wd[t], bd[t], ws[t], bs[t], wu[t], bu[t]))
  ha = m[1]
  hp = 0.5 * (m[0] + m[2])
  hap[...] = ha
  hpp[...] = hp
  stats[0, 0, 0] = jnp.sum(ha, axis=0)
  stats[0, 0, 1] = jnp.sum(ha * ha, axis=0)
  stats[0, 1, 0] = jnp.sum(hp, axis=0)
  stats[0, 1, 1] = jnp.sum(hp * hp, axis=0)


def _bn_act(x, st, gamma, beta):
  mu = jnp.sum(st[:, 0], axis=0) / N
  var = jnp.sum(st[:, 1], axis=0) / N - mu * mu
  xn = (x - mu) / jnp.sqrt(var + EPS) * gamma + beta
  return jnp.where(xn >= 0, xn, 0.01 * xn)


def _dense_b_kernel(hap, hpp, stats, ga, ba, gp, bp, ha_o, hp_o):
  st = stats[...]
  ha_o[...] = _bn_act(hap[...], st[:, 0], ga[0], ba[0])
  hp_o[...] = _bn_act(hpp[...], st[:, 1], gp[0], bp[0])


def _dense_b2_kernel(hap, hpp, stats, ga, ba, gp, bp, wpa, bpa, wpp, bpp,
                     out_o):
  st = stats[...]
  ha = _bn_act(hap[...], st[:, 0], ga[0], ba[0])
  hp = _bn_act(hpp[...], st[:, 1], gp[0], bp[0])
  out_o[0] = jnp.dot(ha, wpa[...], preferred_element_type=jnp.float32) + bpa[0]
  out_o[1] = jnp.dot(hp, wpp[...], preferred_element_type=jnp.float32) + bpp[0]


def _row_spec(shape):
  nd = len(shape)
  blk = tuple(shape[:-2]) + (GB, shape[-1])
  def im(i):
    return tuple([0] * (nd - 2) + [i, 0])
  return pl.BlockSpec(blk, im)


def _full_spec(shape):
  nd = len(shape)
  return pl.BlockSpec(tuple(shape), lambda i, nd=nd: (0,) * nd)


def _dense_a(xa, xp, sums, cnt, wd, bd, ws, bs, wu, bu):
  in_arrs = (xa, xp, sums, cnt, wd, bd, ws, bs, wu, bu)
  in_specs = [
      _row_spec(xa.shape), _row_spec(xp.shape),
      _row_spec(sums.shape), _row_spec(cnt.shape),
      _full_spec(wd.shape), _full_spec(bd.shape),
      _full_spec(ws.shape), _full_spec(bs.shape),
      _full_spec(wu.shape), _full_spec(bu.shape),
  ]
  out_shape = (
      jax.ShapeDtypeStruct((N, H), jnp.float32),
      jax.ShapeDtypeStruct((N, H), jnp.float32),
      jax.ShapeDtypeStruct((G, 2, 2, H), jnp.float32),
  )
  out_specs = (
      _row_spec((N, H)), _row_spec((N, H)),
      pl.BlockSpec((1, 2, 2, H), lambda i: (i, 0, 0, 0)),
  )
  return pl.pallas_call(
      _dense_a_kernel, grid=(G,), in_specs=in_specs,
      out_specs=out_specs, out_shape=out_shape)(*in_arrs)


def _dense_b(hap, hpp, stats, ga, ba, gp, bp):
  in_specs = [
      _row_spec((N, H)), _row_spec((N, H)), _full_spec(stats.shape),
      _full_spec(ga.shape), _full_spec(ba.shape),
      _full_spec(gp.shape), _full_spec(bp.shape),
  ]
  out_shape = (
      jax.ShapeDtypeStruct((N, H), jnp.float32),
      jax.ShapeDtypeStruct((N, H), jnp.float32),
  )
  out_specs = (_row_spec((N, H)), _row_spec((N, H)))
  return pl.pallas_call(
      _dense_b_kernel, grid=(G,), in_specs=in_specs,
      out_specs=out_specs, out_shape=out_shape)(hap, hpp, stats, ga, ba, gp, bp)


def _dense_b2(hap, hpp, stats, ga, ba, gp, bp, wpa, bpa, wpp, bpp):
  in_specs = [
      _row_spec((N, H)), _row_spec((N, H)), _full_spec(stats.shape),
      _full_spec(ga.shape), _full_spec(ba.shape),
      _full_spec(gp.shape), _full_spec(bp.shape),
      _full_spec(wpa.shape), _full_spec(bpa.shape),
      _full_spec(wpp.shape), _full_spec(bpp.shape),
  ]
  out_shape = jax.ShapeDtypeStruct((2, N, L_OUT), jnp.float32)
  out_specs = pl.BlockSpec((2, GB, L_OUT), lambda i: (0, i, 0))
  return pl.pallas_call(
      _dense_b2_kernel, grid=(G,), in_specs=in_specs,
      out_specs=out_specs, out_shape=out_shape)(
          hap, hpp, stats, ga, ba, gp, bp, wpa, bpa, wpp, bpp)


def _prep_edges(ei):
  pad = NW * EP - E
  s = jnp.concatenate([ei[0], jnp.zeros((pad,), jnp.int32)])
  d = jnp.concatenate([ei[1], jnp.full((pad,), N, jnp.int32)])
  return s.reshape(NW, NCH, C), d.reshape(NW, NCH, C)


def _stack_sage(p_layer):
  wd = jnp.stack([p_layer[k]["dst"]["W"] for k in ("writes", "rev", "cites")])
  bd = jnp.stack([p_layer[k]["dst"]["b"][None] for k in ("writes", "rev", "cites")])
  ws = jnp.stack([p_layer[k]["src"]["W"] for k in ("writes", "rev", "cites")])
  bs = jnp.stack([p_layer[k]["src"]["b"][None] for k in ("writes", "rev", "cites")])
  wu = jnp.stack([p_layer[k]["upd"]["W"] for k in ("writes", "rev", "cites")])
  bu = jnp.stack([p_layer[k]["upd"]["b"][None] for k in ("writes", "rev", "cites")])
  return wd, bd, ws, bs, wu, bu


def kernel(x_author, x_paper, params, ei_writes, ei_rev, ei_cites):
  sw, dw = _prep_edges(ei_writes)
  sr, dr = _prep_edges(ei_rev)
  sc, dc = _prep_edges(ei_cites)
  src = jnp.stack([sw, sr, sc])
  dst = jnp.stack([dw, dr, dc])

  sums1, cnt = _make_seg_call(True)(x_author, x_paper, src, dst)

  w1 = _stack_sage(params["l1"])
  bn1 = params["bn1"]
  hap1, hpp1, st1 = _dense_a(x_author, x_paper, sums1, cnt, *w1)
  ha1, hp1 = _dense_b(hap1, hpp1, st1,
                      bn1["author"]["gamma"][None], bn1["author"]["beta"][None],
                      bn1["paper"]["gamma"][None], bn1["paper"]["beta"][None])

  (sums2,) = _make_seg_call(False)(ha1, hp1, src, dst)

  w2 = _stack_sage(params["l2"])
  bn2 = params["bn2"]
  hap2, hpp2, st2 = _dense_a(ha1, hp1, sums2, cnt, *w2)
  out = _dense_b2(hap2, hpp2, st2,
                  bn2["author"]["gamma"][None], bn2["author"]["beta"][None],
                  bn2["paper"]["gamma"][None], bn2["paper"]["beta"][None],
                  params["post"]["author"]["W"], params["post"]["author"]["b"][None],
                  params["post"]["paper"]["W"], params["post"]["paper"]["b"][None])
  return out
